# baseline (speedup 1.0000x reference)
import functools

import jax
import jax.numpy as jnp
import numpy as np
from jax import lax
from jax.experimental import pallas as pl
from jax.experimental.pallas import tpu as pltpu
from jax.experimental.pallas import tpu_sc as plsc

_SPATIAL_LEVELS = 24
_TEMPORAL_LEVELS = 24
_F = 2
_TABLE = 65536
_LEVELS = _SPATIAL_LEVELS + 3 * _TEMPORAL_LEVELS
_ENC_DIM = _LEVELS * _F
_IN_DIM = 13
_HID = 64

_NW = 32
_LPW = _LEVELS // _NW
_CH = 2048
_NCHUNK = 21
_NPAD = _CH * _NCHUNK

_P1 = np.int32(np.uint32(2654435761).view(np.int32))
_P2 = np.int32(805459861)


def _resolutions(n_levels, base=16.0, max_res=4096.0):
    g = np.exp(np.log(max_res / base) / max(n_levels - 1, 1))
    return [float(base * g**l) for l in range(n_levels)]


_SRES = _resolutions(_SPATIAL_LEVELS)
_TRES = _resolutions(_TEMPORAL_LEVELS)
_RES_LIST = _SRES + [_TRES[tl] for tl in range(_TEMPORAL_LEVELS) for _ in range(3)]


def _encode_body(spat_hbm, temp_hbm, coordsp_hbm, res_hbm, enc_hbm, table_v, cbuf, ebuf, res_v, sem):
    wid = lax.axis_index("s") * 2 + lax.axis_index("c")
    pltpu.sync_copy(res_hbm, res_v)

    for k in range(_LPW):
        level = wid * _LPW + k
        pi = jnp.where(level < _SPATIAL_LEVELS,
                       0, 1 + lax.rem(level - _SPATIAL_LEVELS, 3))
        @pl.when(level < _SPATIAL_LEVELS)
        def _():
            pltpu.sync_copy(spat_hbm.at[pl.ds(level * _TABLE, _TABLE)],
                            table_v)

        @pl.when(level >= _SPATIAL_LEVELS)
        def _():
            pltpu.sync_copy(
                temp_hbm.at[pl.ds((level - _SPATIAL_LEVELS) * _TABLE, _TABLE)],
                table_v)

        resv = plsc.load_gather(res_v, [jnp.full((16,), level, jnp.int32)])
        row0 = level * 2

        r0 = jnp.where(pi == 3, 1, 0)
        r1 = jnp.where(pi <= 1, 1, 2)
        r2 = jnp.where(pi == 0, 2, 3)

        def chunk_body(c, _, rows=(r0, r1, r2), resv=resv, row0=row0):
            base = c * _CH
            for j in range(3):
                pltpu.sync_copy(
                    coordsp_hbm.at[pl.ds(rows[j] * _NPAD + base, _CH)],
                    cbuf.at[pl.ds(j * _CH, _CH)])

            @plsc.parallel_loop(0, _CH, 16, unroll=4)
            def group_body(lb, resv=resv):
                cx = cbuf[pl.ds(lb, 16)] * resv
                cy = cbuf[pl.ds(_CH + lb, 16)] * resv
                cz = cbuf[pl.ds(2 * _CH + lb, 16)] * resv
                xi = cx.astype(jnp.int32)
                yi = cy.astype(jnp.int32)
                zi = cz.astype(jnp.int32)
                fx = cx - xi.astype(jnp.float32)
                fy = cy - yi.astype(jnp.float32)
                fz = cz - zi.astype(jnp.float32)
                a = (xi, xi + 1)
                b = (yi * _P1, yi * _P1 + _P1)
                cc = (zi * _P2, zi * _P2 + _P2)
                wxs = (1.0 - fx, fx)
                wys = (1.0 - fy, fy)
                wzs = (1.0 - fz, fz)
                acc0 = jnp.zeros((16,), jnp.float32)
                acc1 = jnp.zeros((16,), jnp.float32)
                for dx in range(2):
                    for dy in range(2):
                        ab = a[dx] ^ b[dy]
                        wxy = wxs[dx] * wys[dy]
                        for dz in range(2):
                            h = (ab ^ cc[dz]) & jnp.int32(_TABLE - 1)
                            w = wxy * wzs[dz]
                            t = plsc.load_gather(table_v, [h])
                            f0 = plsc.bitcast(t << 16, jnp.float32)
                            f1 = plsc.bitcast(t & jnp.int32(-65536), jnp.float32)
                            acc0 = acc0 + w * f0
                            acc1 = acc1 + w * f1
                ebuf[pl.ds(lb, 16)] = acc0
                ebuf[pl.ds(_CH + lb, 16)] = acc1

            pltpu.sync_copy(ebuf.at[pl.ds(0, _CH)],
                            enc_hbm.at[pl.ds(row0 * _NPAD + base, _CH)])
            pltpu.sync_copy(ebuf.at[pl.ds(_CH, _CH)],
                            enc_hbm.at[pl.ds((row0 + 1) * _NPAD + base, _CH)])
            return 0

        lax.fori_loop(0, _NCHUNK, chunk_body, 0)


@functools.cache
def _encode_sc():
    return pl.kernel(
        _encode_body,
        out_type=jax.ShapeDtypeStruct((_ENC_DIM * _NPAD,), jnp.float32),
        mesh=plsc.VectorSubcoreMesh(core_axis_name="c", subcore_axis_name="s",
                                    num_cores=2, num_subcores=16),
        compiler_params=pltpu.CompilerParams(needs_layout_passes=False),
        scratch_types=[
            pltpu.VMEM((_TABLE,), jnp.int32),
            pltpu.VMEM((3 * _CH,), jnp.float32),
            pltpu.VMEM((2 * _CH,), jnp.float32),
            pltpu.VMEM((_LEVELS,), jnp.float32),
            pltpu.SemaphoreType.DMA,
        ],
    )


_TN = 2048
_B = 4


def _ln(h, g_row, be_row, mean_mat):
    mu = jax.lax.dot_general(h, mean_mat, (((1,), (0,)), ((), ())),
                             preferred_element_type=jnp.float32)
    ex2 = jax.lax.dot_general(h * h, mean_mat, (((1,), (0,)), ((), ())),
                              preferred_element_type=jnp.float32)
    var = ex2 - mu * mu
    return (h - mu) * (lax.rsqrt(var + 1e-5) * g_row) + be_row


def _mlp_body(enc_ref, nf_ref, w1e_ref, w1n_ref, b1_ref, g1_ref, be1_ref,
              w2_ref, b2_ref, g2_ref, be2_ref, out_ref):
    mean_mat = jnp.full((_HID, _HID), 1.0 / _HID, jnp.float32)
    h1e = lax.dot_general(
        enc_ref[...], w1e_ref[...],
        (((0,), (0,)), ((), ())), preferred_element_type=jnp.float32)
    for b in range(_B):
        h = h1e + jnp.dot(nf_ref[b], w1n_ref[...],
                          preferred_element_type=jnp.float32) + b1_ref[...]
        h = _ln(h, g1_ref[...], be1_ref[...], mean_mat)
        h = jnp.maximum(h, 0.0)
        h = jnp.dot(h, w2_ref[...],
                    preferred_element_type=jnp.float32) + b2_ref[...]
        out_ref[b] = _ln(h, g2_ref[...], be2_ref[...], mean_mat)


def kernel(node_features, coordinates, spatial_tables, temporal_tables,
           W1, b1, g1, be1, W2, b2, g2, be2):
    B, N = node_features.shape[0], node_features.shape[1]

    def _pack(t):
        return lax.bitcast_convert_type(t.astype(jnp.bfloat16),
                                        jnp.int32).reshape(-1)

    packed_s = _pack(spatial_tables)
    packed_t = _pack(temporal_tables)

    coordsp = jnp.pad(coordinates.T, ((0, 0), (0, _NPAD - N)))
    res_arr = jnp.asarray(_RES_LIST, jnp.float32)

    enc_t = _encode_sc()(packed_s, packed_t, coordsp.reshape(-1),
                         res_arr).reshape(_ENC_DIM, _NPAD)

    w1e, w1n = W1[:_ENC_DIM], W1[_ENC_DIM:]
    nj = (N + _TN - 1) // _TN
    out = pl.pallas_call(
        _mlp_body,
        out_shape=jax.ShapeDtypeStruct((B, N, _HID), jnp.float32),
        grid=(nj,),
        in_specs=[
            pl.BlockSpec((_ENC_DIM, _TN), lambda j: (0, j)),
            pl.BlockSpec((_B, _TN, _IN_DIM), lambda j: (0, j, 0)),
            pl.BlockSpec((_ENC_DIM, _HID), lambda j: (0, 0)),
            pl.BlockSpec((_IN_DIM, _HID), lambda j: (0, 0)),
            pl.BlockSpec((1, _HID), lambda j: (0, 0)),
            pl.BlockSpec((1, _HID), lambda j: (0, 0)),
            pl.BlockSpec((1, _HID), lambda j: (0, 0)),
            pl.BlockSpec((_HID, _HID), lambda j: (0, 0)),
            pl.BlockSpec((1, _HID), lambda j: (0, 0)),
            pl.BlockSpec((1, _HID), lambda j: (0, 0)),
            pl.BlockSpec((1, _HID), lambda j: (0, 0)),
        ],
        out_specs=pl.BlockSpec((_B, _TN, _HID), lambda j: (0, j, 0)),
    )(enc_t, node_features, w1e, w1n,
      b1.reshape(1, -1), g1.reshape(1, -1), be1.reshape(1, -1),
      W2, b2.reshape(1, -1), g2.reshape(1, -1), be2.reshape(1, -1))
    return out


# speedup vs baseline: 1.1435x; 1.1435x over previous
import functools

import jax
import jax.numpy as jnp
import numpy as np
from jax import lax
from jax.experimental import pallas as pl
from jax.experimental.pallas import tpu as pltpu
from jax.experimental.pallas import tpu_sc as plsc

_SPATIAL_LEVELS = 24
_TEMPORAL_LEVELS = 24
_F = 2
_TABLE = 65536
_LEVELS = _SPATIAL_LEVELS + 3 * _TEMPORAL_LEVELS
_ENC_DIM = _LEVELS * _F
_IN_DIM = 13
_HID = 64

_NW = 32
_LPW = _LEVELS // _NW
_CH = 2048
_NCHUNK = 21
_NPAD = _CH * _NCHUNK

_P1 = np.int32(np.uint32(2654435761).view(np.int32))
_P2 = np.int32(805459861)


def _resolutions(n_levels, base=16.0, max_res=4096.0):
    g = np.exp(np.log(max_res / base) / max(n_levels - 1, 1))
    return [float(base * g**l) for l in range(n_levels)]


_SRES = _resolutions(_SPATIAL_LEVELS)
_TRES = _resolutions(_TEMPORAL_LEVELS)
_RES_LIST = _SRES + [_TRES[tl] for tl in range(_TEMPORAL_LEVELS) for _ in range(3)]


def _encode_body(spat_hbm, temp_hbm, coordsp_hbm, res_hbm, enc_hbm, table_v, cbuf, ebuf, res_v, sem):
    wid = lax.axis_index("s") * 2 + lax.axis_index("c")
    pltpu.sync_copy(res_hbm, res_v)

    for k in range(_LPW):
        level = wid * _LPW + k
        pi = jnp.where(level < _SPATIAL_LEVELS,
                       0, 1 + lax.rem(level - _SPATIAL_LEVELS, 3))
        @pl.when(level < _SPATIAL_LEVELS)
        def _():
            pltpu.sync_copy(spat_hbm.at[pl.ds(level * _TABLE, _TABLE)],
                            table_v)

        @pl.when(level >= _SPATIAL_LEVELS)
        def _():
            pltpu.sync_copy(
                temp_hbm.at[pl.ds((level - _SPATIAL_LEVELS) * _TABLE, _TABLE)],
                table_v)

        resv = plsc.load_gather(res_v, [jnp.full((16,), level, jnp.int32)])
        row0 = level * 2

        def chunk_body(c, _, resv=resv, row0=row0, pi=pi):
            pltpu.sync_copy(
                coordsp_hbm.at[pl.ds((pi * _NCHUNK + c) * 3 * _CH, 3 * _CH)],
                cbuf)

            @plsc.parallel_loop(0, _CH, 16, unroll=2)
            def group_body(lb, resv=resv):
                cx = cbuf[pl.ds(lb, 16)] * resv
                cy = cbuf[pl.ds(_CH + lb, 16)] * resv
                cz = cbuf[pl.ds(2 * _CH + lb, 16)] * resv
                xi = cx.astype(jnp.int32)
                yi = cy.astype(jnp.int32)
                zi = cz.astype(jnp.int32)
                fx = cx - xi.astype(jnp.float32)
                fy = cy - yi.astype(jnp.float32)
                fz = cz - zi.astype(jnp.float32)
                a = (xi, xi + 1)
                b = (yi * _P1, yi * _P1 + _P1)
                cc = (zi * _P2, zi * _P2 + _P2)
                wxs = (1.0 - fx, fx)
                wys = (1.0 - fy, fy)
                wzs = (1.0 - fz, fz)
                acc0 = jnp.zeros((16,), jnp.float32)
                acc1 = jnp.zeros((16,), jnp.float32)
                for dx in range(2):
                    for dy in range(2):
                        ab = a[dx] ^ b[dy]
                        wxy = wxs[dx] * wys[dy]
                        for dz in range(2):
                            h = (ab ^ cc[dz]) & jnp.int32(_TABLE - 1)
                            w = wxy * wzs[dz]
                            t = plsc.load_gather(table_v, [h])
                            f0 = plsc.bitcast(t << 16, jnp.float32)
                            f1 = plsc.bitcast(t & jnp.int32(-65536), jnp.float32)
                            acc0 = acc0 + w * f0
                            acc1 = acc1 + w * f1
                ebuf[pl.ds(lb, 16)] = acc0
                ebuf[pl.ds(_CH + lb, 16)] = acc1

            pltpu.sync_copy(
                ebuf,
                enc_hbm.at[pl.ds((c * _ENC_DIM + row0) * _CH, 2 * _CH)])
            return 0

        lax.fori_loop(0, _NCHUNK, chunk_body, 0)


@functools.cache
def _encode_sc():
    return pl.kernel(
        _encode_body,
        out_type=jax.ShapeDtypeStruct((_ENC_DIM * _NPAD,), jnp.float32),
        mesh=plsc.VectorSubcoreMesh(core_axis_name="c", subcore_axis_name="s",
                                    num_cores=2, num_subcores=16),
        compiler_params=pltpu.CompilerParams(needs_layout_passes=False),
        scratch_types=[
            pltpu.VMEM((_TABLE,), jnp.int32),
            pltpu.VMEM((3 * _CH,), jnp.float32),
            pltpu.VMEM((2 * _CH,), jnp.float32),
            pltpu.VMEM((_LEVELS,), jnp.float32),
            pltpu.SemaphoreType.DMA,
        ],
    )


_TN = 2048
_B = 4


def _ln(h, g_row, be_row, mean_mat):
    mu = jax.lax.dot_general(h, mean_mat, (((1,), (0,)), ((), ())),
                             preferred_element_type=jnp.float32)
    ex2 = jax.lax.dot_general(h * h, mean_mat, (((1,), (0,)), ((), ())),
                              preferred_element_type=jnp.float32)
    var = ex2 - mu * mu
    return (h - mu) * (lax.rsqrt(var + 1e-5) * g_row) + be_row


def _mlp_body(enc_ref, nf_ref, w1e_ref, w1n_ref, b1_ref, g1_ref, be1_ref,
              w2_ref, b2_ref, g2_ref, be2_ref, out_ref):
    mean_mat = jnp.full((_HID, _HID), 1.0 / _HID, jnp.float32)
    h1e = lax.dot_general(
        enc_ref[0], w1e_ref[...],
        (((0,), (0,)), ((), ())), preferred_element_type=jnp.float32)
    for b in range(_B):
        h = h1e + jnp.dot(nf_ref[b], w1n_ref[...],
                          preferred_element_type=jnp.float32) + b1_ref[...]
        h = _ln(h, g1_ref[...], be1_ref[...], mean_mat)
        h = jnp.maximum(h, 0.0)
        h = jnp.dot(h, w2_ref[...],
                    preferred_element_type=jnp.float32) + b2_ref[...]
        out_ref[b] = _ln(h, g2_ref[...], be2_ref[...], mean_mat)


def kernel(node_features, coordinates, spatial_tables, temporal_tables,
           W1, b1, g1, be1, W2, b2, g2, be2):
    B, N = node_features.shape[0], node_features.shape[1]

    def _pack(t):
        return lax.bitcast_convert_type(t.astype(jnp.bfloat16),
                                        jnp.int32).reshape(-1)

    packed_s = _pack(spatial_tables)
    packed_t = _pack(temporal_tables)

    coordsp = jnp.pad(coordinates.T, ((0, 0), (0, _NPAD - N)))
    rows_per_plane = jnp.asarray(
        [[0, 1, 2], [0, 1, 3], [0, 2, 3], [1, 2, 3]], jnp.int32)
    planes = coordsp[rows_per_plane]
    planes = planes.reshape(4, 3, _NCHUNK, _CH).transpose(0, 2, 1, 3)
    res_arr = jnp.asarray(_RES_LIST, jnp.float32)

    enc_t = _encode_sc()(packed_s, packed_t, planes.reshape(-1),
                         res_arr).reshape(_NCHUNK, _ENC_DIM, _CH)

    w1e, w1n = W1[:_ENC_DIM], W1[_ENC_DIM:]
    nj = (N + _TN - 1) // _TN
    out = pl.pallas_call(
        _mlp_body,
        out_shape=jax.ShapeDtypeStruct((B, N, _HID), jnp.float32),
        grid=(nj,),
        in_specs=[
            pl.BlockSpec((1, _ENC_DIM, _TN), lambda j: (j, 0, 0)),
            pl.BlockSpec((_B, _TN, _IN_DIM), lambda j: (0, j, 0)),
            pl.BlockSpec((_ENC_DIM, _HID), lambda j: (0, 0)),
            pl.BlockSpec((_IN_DIM, _HID), lambda j: (0, 0)),
            pl.BlockSpec((1, _HID), lambda j: (0, 0)),
            pl.BlockSpec((1, _HID), lambda j: (0, 0)),
            pl.BlockSpec((1, _HID), lambda j: (0, 0)),
            pl.BlockSpec((_HID, _HID), lambda j: (0, 0)),
            pl.BlockSpec((1, _HID), lambda j: (0, 0)),
            pl.BlockSpec((1, _HID), lambda j: (0, 0)),
            pl.BlockSpec((1, _HID), lambda j: (0, 0)),
        ],
        out_specs=pl.BlockSpec((_B, _TN, _HID), lambda j: (0, j, 0)),
    )(enc_t, node_features, w1e, w1n,
      b1.reshape(1, -1), g1.reshape(1, -1), be1.reshape(1, -1),
      W2, b2.reshape(1, -1), g2.reshape(1, -1), be2.reshape(1, -1))
    return out
